# baseline (speedup 1.0000x reference)
import functools

import jax
import jax.numpy as jnp
from jax import lax
from jax.experimental import pallas as pl
from jax.experimental.pallas import tpu as pltpu
from jax.experimental.pallas import tpu_sc as plsc

B, S, D = 32, 4096, 768
K = S // 4
BS = 2048
NBLK = B * S // BS
RPB = S // BS

_I32_MIN = -2147483648
_I32_TOPMASK = 2147483647


def _stage1_body(x_ref, wt_ref, b_ref, q_ref):
    y = jax.lax.dot_general(wt_ref[...], x_ref[0],
                            (((1,), (1,)), ((), ())),
                            preferred_element_type=jnp.float32)
    l0 = y[0:1, :] + b_ref[0, 0]
    l1 = y[1:2, :] + b_ref[0, 1]
    score = l1 - l0
    m = jnp.maximum(l0, l1)
    lse = m + jnp.log(jnp.exp(l0 - m) + jnp.exp(l1 - m))
    lp0 = l0 - lse
    lp1 = l1 - lse
    ent = -(jnp.exp(lp0) * lp0 + jnp.exp(lp1) * lp1)
    q_ref[0] = jnp.concatenate(
        [score, lp0, lp1, ent, jnp.zeros((4, BS), jnp.float32)], axis=0)


def _sortable_key(score):
    bits = jax.lax.bitcast_convert_type(score, jnp.int32)
    return jnp.where(bits < 0, bits ^ jnp.int32(_I32_TOPMASK), bits)


_UNROLL = 16


HB = B // 2


def _sc_topk_body(q_hbm, act_hbm, lp_hbm, sc_v, p0_v, p1_v, key_v, act_v,
                  lpo_v, sem, sem_p):
    w = lax.axis_index("s") * 2 + lax.axis_index("c")

    @pl.when(w < HB)
    def _():
        _sc_topk_row(q_hbm, act_hbm, lp_hbm, sc_v, p0_v, p1_v, key_v, act_v,
                     lpo_v, sem, sem_p, w)


def _sc_topk_row(q_hbm, act_hbm, lp_hbm, sc_v, p0_v, p1_v, key_v, act_v,
                 lpo_v, sem, sem_p, w):

    copies = [
        pltpu.async_copy(q_hbm.at[2 * w, 0], sc_v.at[pl.ds(0, BS)], sem),
        pltpu.async_copy(q_hbm.at[2 * w + 1, 0], sc_v.at[pl.ds(BS, BS)], sem),
        pltpu.async_copy(q_hbm.at[2 * w, 1], p0_v.at[pl.ds(0, BS)], sem_p),
        pltpu.async_copy(q_hbm.at[2 * w + 1, 1], p0_v.at[pl.ds(BS, BS)],
                         sem_p),
        pltpu.async_copy(q_hbm.at[2 * w, 2], p1_v.at[pl.ds(0, BS)], sem_p),
        pltpu.async_copy(q_hbm.at[2 * w + 1, 2], p1_v.at[pl.ds(BS, BS)],
                         sem_p),
    ]
    copies[0].wait()
    copies[1].wait()

    def keys_body(j, carry):
        base = j * 16 * _UNROLL
        for u in range(_UNROLL):
            off = base + u * 16
            bits = lax.bitcast_convert_type(sc_v[pl.ds(off, 16)], jnp.int32)
            key_v[pl.ds(off, 16)] = jnp.where(
                bits < 0, bits ^ jnp.int32(_I32_TOPMASK), bits)
        return carry

    lax.fori_loop(0, S // (16 * _UNROLL), keys_body, jnp.int32(0))

    def count_matching(pred):
        def body(j, acc):
            base = j * 16 * _UNROLL
            for u in range(_UNROLL):
                k16 = key_v[pl.ds(base + u * 16, 16)]
                acc = acc + jnp.where(pred(k16, base + u * 16), 1, 0)
            return acc

        acc = lax.fori_loop(0, S // (16 * _UNROLL), body,
                            jnp.zeros((16,), jnp.int32))
        return jnp.sum(acc)

    def bit_body(i, carry):
        t_u, c_ge = carry
        cand = t_u | jnp.left_shift(jnp.int32(1), 31 - i)
        scand = cand ^ jnp.int32(_I32_MIN)
        cnt = count_matching(lambda k16, off: k16 >= scand)
        ok = cnt >= K
        return jnp.where(ok, cand, t_u), jnp.where(ok, cnt, c_ge)

    t_u, c_ge = lax.fori_loop(0, 32, bit_body,
                              (jnp.int32(0), jnp.int32(S)))
    thr = t_u ^ jnp.int32(_I32_MIN)

    c_gt = count_matching(lambda k16, off: k16 > thr)
    need = K - c_gt

    iota16 = lax.iota(jnp.int32, 16)

    def idx_descend():
        def idx_body(i, t_j):
            cand = t_j | jnp.left_shift(jnp.int32(1), 11 - i)
            cnt = count_matching(
                lambda k16, off: (k16 == thr) & (iota16 + off <= cand))
            return jnp.where(cnt <= need, cand, t_j)

        return lax.fori_loop(0, 12, idx_body, jnp.int32(0))

    t_j = lax.cond(c_ge - c_gt == need, lambda: jnp.int32(S), idx_descend)

    copies[2].wait()
    copies[3].wait()
    copies[4].wait()
    copies[5].wait()

    def fin_body(j, carry):
        base = j * 16 * _UNROLL
        for u in range(_UNROLL):
            off = base + u * 16
            k16 = key_v[pl.ds(off, 16)]
            m = (k16 > thr) | ((k16 == thr) & (iota16 + off <= t_j))
            act_v[pl.ds(off, 16)] = jnp.where(m, 1, 0)
            lpo_v[pl.ds(off, 16)] = jnp.where(
                m, p1_v[pl.ds(off, 16)], p0_v[pl.ds(off, 16)])
        return carry

    lax.fori_loop(0, S // (16 * _UNROLL), fin_body, jnp.int32(0))

    out_copies = [
        pltpu.async_copy(act_v, act_hbm.at[w], sem),
        pltpu.async_copy(lpo_v, lp_hbm.at[w], sem),
    ]
    out_copies[0].wait()
    out_copies[1].wait()


def _stage2_body(q_ref, act_ref, lp_ref, ent_ref):
    score = q_ref[:, 0, :].reshape(B, S)
    lp0 = q_ref[:, 1, :].reshape(B, S)
    lp1 = q_ref[:, 2, :].reshape(B, S)
    ent_ref[...] = q_ref[:, 3, :].reshape(B, S)

    key = _sortable_key(score)

    def bit_step(i, t_u):
        cand = t_u | jnp.left_shift(jnp.int32(1), 31 - i)
        scand = cand ^ jnp.int32(_I32_MIN)
        cnt = jnp.sum((key >= scand).astype(jnp.int32), axis=1, keepdims=True)
        return jnp.where(cnt >= K, cand, t_u)

    t_u = jax.lax.fori_loop(0, 32, bit_step, jnp.zeros((B, 1), jnp.int32))
    thr = t_u ^ jnp.int32(_I32_MIN)

    gt = key > thr
    eq = key == thr
    c_gt = jnp.sum(gt.astype(jnp.int32), axis=1, keepdims=True)
    need = K - c_gt

    idx = jax.lax.broadcasted_iota(jnp.int32, (B, S), 1)

    def idx_step(i, t_j):
        cand = t_j | jnp.left_shift(jnp.int32(1), 11 - i)
        cnt = jnp.sum((eq & (idx <= cand)).astype(jnp.int32), axis=1,
                      keepdims=True)
        return jnp.where(cnt <= need, cand, t_j)

    t_j = jax.lax.fori_loop(0, 12, idx_step, jnp.zeros((B, 1), jnp.int32))

    mask = gt | (eq & (idx <= t_j))
    act_ref[...] = mask.astype(jnp.int32)
    lp_ref[...] = jnp.where(mask, lp1, lp0)


@jax.jit
def kernel(x, W, b):
    x3 = x.reshape(NBLK, BS, D)
    hblk = NBLK // 2
    fdef = jax.ShapeDtypeStruct((HB, S), jnp.float32)
    sc_topk = functools.partial(
        pl.kernel,
        mesh=plsc.VectorSubcoreMesh(core_axis_name="c", subcore_axis_name="s"),
        out_type=[jax.ShapeDtypeStruct((HB, S), jnp.int32), fdef],
        compiler_params=pltpu.CompilerParams(needs_layout_passes=False),
        scratch_types=[
            pltpu.VMEM((S,), jnp.float32),
            pltpu.VMEM((S,), jnp.float32),
            pltpu.VMEM((S,), jnp.float32),
            pltpu.VMEM((S,), jnp.int32),
            pltpu.VMEM((S,), jnp.int32),
            pltpu.VMEM((S,), jnp.float32),
            pltpu.SemaphoreType.DMA,
            pltpu.SemaphoreType.DMA,
        ],
    )(_sc_topk_body)

    halves = []
    for h in (0, 1):
        q_h = pl.pallas_call(
            _stage1_body,
            grid=(hblk,),
            in_specs=[
                pl.BlockSpec((1, BS, D), lambda i, _h=h: (i + _h * hblk, 0, 0)),
                pl.BlockSpec((2, D), lambda i: (0, 0)),
                pl.BlockSpec((1, 2), lambda i: (0, 0)),
            ],
            out_specs=pl.BlockSpec((1, 8, BS), lambda i: (i, 0, 0)),
            out_shape=jax.ShapeDtypeStruct((hblk, 8, BS), jnp.float32),
            compiler_params=pltpu.CompilerParams(
                dimension_semantics=("arbitrary",)),
        )(x3, W.T, b.reshape(1, 2))
        halves.append(q_h)

    outs = [sc_topk(q_h) for q_h in halves]
    actions = jnp.concatenate([o[0] for o in outs], axis=0)
    log_probs = jnp.concatenate([o[1] for o in outs], axis=0)
    ent = jnp.concatenate(
        [q_h[:, 3, :].reshape(HB, S) for q_h in halves], axis=0)

    topk_log_probs = jnp.zeros((B, S), jnp.float32)
    return (actions, topk_log_probs, log_probs, ent)


# speedup vs baseline: 1.0391x; 1.0391x over previous
import functools

import jax
import jax.numpy as jnp
from jax import lax
from jax.experimental import pallas as pl
from jax.experimental.pallas import tpu as pltpu
from jax.experimental.pallas import tpu_sc as plsc

B, S, D = 32, 4096, 768
K = S // 4
BS = 2048
NBLK = B * S // BS
RPB = S // BS

_I32_MIN = -2147483648
_I32_TOPMASK = 2147483647


def _stage1_body(x_ref, wt_ref, b_ref, q_ref):
    y = jax.lax.dot_general(wt_ref[...], x_ref[0],
                            (((1,), (1,)), ((), ())),
                            preferred_element_type=jnp.float32)
    l0 = y[0:1, :] + b_ref[0, 0]
    l1 = y[1:2, :] + b_ref[0, 1]
    score = l1 - l0
    m = jnp.maximum(l0, l1)
    lse = m + jnp.log(jnp.exp(l0 - m) + jnp.exp(l1 - m))
    lp0 = l0 - lse
    lp1 = l1 - lse
    ent = -(jnp.exp(lp0) * lp0 + jnp.exp(lp1) * lp1)
    q_ref[0] = jnp.concatenate(
        [score, lp0, lp1, ent, jnp.zeros((4, BS), jnp.float32)], axis=0)


def _sortable_key(score):
    bits = jax.lax.bitcast_convert_type(score, jnp.int32)
    return jnp.where(bits < 0, bits ^ jnp.int32(_I32_TOPMASK), bits)


_UNROLL = 16


def _sc_topk_body(q_hbm, act_hbm, lp_hbm, sc_v, p0_v, p1_v, key_v, act_v,
                  lpo_v, sem, sem_p):
    w = lax.axis_index("s") * 2 + lax.axis_index("c")

    copies = [
        pltpu.async_copy(q_hbm.at[2 * w, 0], sc_v.at[pl.ds(0, BS)], sem),
        pltpu.async_copy(q_hbm.at[2 * w + 1, 0], sc_v.at[pl.ds(BS, BS)], sem),
        pltpu.async_copy(q_hbm.at[2 * w, 1], p0_v.at[pl.ds(0, BS)], sem_p),
        pltpu.async_copy(q_hbm.at[2 * w + 1, 1], p0_v.at[pl.ds(BS, BS)],
                         sem_p),
        pltpu.async_copy(q_hbm.at[2 * w, 2], p1_v.at[pl.ds(0, BS)], sem_p),
        pltpu.async_copy(q_hbm.at[2 * w + 1, 2], p1_v.at[pl.ds(BS, BS)],
                         sem_p),
    ]
    copies[0].wait()
    copies[1].wait()

    def keys_body(j, carry):
        base = j * 16 * _UNROLL
        for u in range(_UNROLL):
            off = base + u * 16
            bits = lax.bitcast_convert_type(sc_v[pl.ds(off, 16)], jnp.int32)
            key_v[pl.ds(off, 16)] = jnp.where(
                bits < 0, bits ^ jnp.int32(_I32_TOPMASK), bits)
        return carry

    lax.fori_loop(0, S // (16 * _UNROLL), keys_body, jnp.int32(0))

    def count_matching(pred):
        def body(j, acc):
            base = j * 16 * _UNROLL
            for u in range(_UNROLL):
                k16 = key_v[pl.ds(base + u * 16, 16)]
                acc = acc + jnp.where(pred(k16, base + u * 16), 1, 0)
            return acc

        acc = lax.fori_loop(0, S // (16 * _UNROLL), body,
                            jnp.zeros((16,), jnp.int32))
        return jnp.sum(acc)

    def bit_body(i, carry):
        t_u, c_ge = carry
        cand = t_u | jnp.left_shift(jnp.int32(1), 31 - i)
        scand = cand ^ jnp.int32(_I32_MIN)
        cnt = count_matching(lambda k16, off: k16 >= scand)
        ok = cnt >= K
        return jnp.where(ok, cand, t_u), jnp.where(ok, cnt, c_ge)

    t_u, c_ge = lax.fori_loop(0, 32, bit_body,
                              (jnp.int32(0), jnp.int32(S)))
    thr = t_u ^ jnp.int32(_I32_MIN)

    c_gt = count_matching(lambda k16, off: k16 > thr)
    need = K - c_gt

    iota16 = lax.iota(jnp.int32, 16)

    def idx_descend():
        def idx_body(i, t_j):
            cand = t_j | jnp.left_shift(jnp.int32(1), 11 - i)
            cnt = count_matching(
                lambda k16, off: (k16 == thr) & (iota16 + off <= cand))
            return jnp.where(cnt <= need, cand, t_j)

        return lax.fori_loop(0, 12, idx_body, jnp.int32(0))

    t_j = lax.cond(c_ge - c_gt == need, lambda: jnp.int32(S), idx_descend)

    copies[2].wait()
    copies[3].wait()
    copies[4].wait()
    copies[5].wait()

    def fin_body(j, carry):
        base = j * 16 * _UNROLL
        for u in range(_UNROLL):
            off = base + u * 16
            k16 = key_v[pl.ds(off, 16)]
            m = (k16 > thr) | ((k16 == thr) & (iota16 + off <= t_j))
            act_v[pl.ds(off, 16)] = jnp.where(m, 1, 0)
            lpo_v[pl.ds(off, 16)] = jnp.where(
                m, p1_v[pl.ds(off, 16)], p0_v[pl.ds(off, 16)])
        return carry

    lax.fori_loop(0, S // (16 * _UNROLL), fin_body, jnp.int32(0))

    out_copies = [
        pltpu.async_copy(act_v, act_hbm.at[w], sem),
        pltpu.async_copy(lpo_v, lp_hbm.at[w], sem),
    ]
    out_copies[0].wait()
    out_copies[1].wait()


def _stage2_body(q_ref, act_ref, lp_ref, ent_ref):
    score = q_ref[:, 0, :].reshape(B, S)
    lp0 = q_ref[:, 1, :].reshape(B, S)
    lp1 = q_ref[:, 2, :].reshape(B, S)
    ent_ref[...] = q_ref[:, 3, :].reshape(B, S)

    key = _sortable_key(score)

    def bit_step(i, t_u):
        cand = t_u | jnp.left_shift(jnp.int32(1), 31 - i)
        scand = cand ^ jnp.int32(_I32_MIN)
        cnt = jnp.sum((key >= scand).astype(jnp.int32), axis=1, keepdims=True)
        return jnp.where(cnt >= K, cand, t_u)

    t_u = jax.lax.fori_loop(0, 32, bit_step, jnp.zeros((B, 1), jnp.int32))
    thr = t_u ^ jnp.int32(_I32_MIN)

    gt = key > thr
    eq = key == thr
    c_gt = jnp.sum(gt.astype(jnp.int32), axis=1, keepdims=True)
    need = K - c_gt

    idx = jax.lax.broadcasted_iota(jnp.int32, (B, S), 1)

    def idx_step(i, t_j):
        cand = t_j | jnp.left_shift(jnp.int32(1), 11 - i)
        cnt = jnp.sum((eq & (idx <= cand)).astype(jnp.int32), axis=1,
                      keepdims=True)
        return jnp.where(cnt <= need, cand, t_j)

    t_j = jax.lax.fori_loop(0, 12, idx_step, jnp.zeros((B, 1), jnp.int32))

    mask = gt | (eq & (idx <= t_j))
    act_ref[...] = mask.astype(jnp.int32)
    lp_ref[...] = jnp.where(mask, lp1, lp0)


@jax.jit
def kernel(x, W, b):
    x3 = x.reshape(NBLK, BS, D)
    q = pl.pallas_call(
        _stage1_body,
        grid=(NBLK,),
        in_specs=[
            pl.BlockSpec((1, BS, D), lambda i: (i, 0, 0)),
            pl.BlockSpec((2, D), lambda i: (0, 0)),
            pl.BlockSpec((1, 2), lambda i: (0, 0)),
        ],
        out_specs=pl.BlockSpec((1, 8, BS), lambda i: (i, 0, 0)),
        out_shape=jax.ShapeDtypeStruct((NBLK, 8, BS), jnp.float32),
        compiler_params=pltpu.CompilerParams(
            dimension_semantics=("arbitrary",)),
    )(x3, W.T, b.reshape(1, 2))

    fdef = jax.ShapeDtypeStruct((B, S), jnp.float32)
    sc_topk = functools.partial(
        pl.kernel,
        mesh=plsc.VectorSubcoreMesh(core_axis_name="c", subcore_axis_name="s"),
        out_type=[jax.ShapeDtypeStruct((B, S), jnp.int32), fdef],
        compiler_params=pltpu.CompilerParams(needs_layout_passes=False),
        scratch_types=[
            pltpu.VMEM((S,), jnp.float32),
            pltpu.VMEM((S,), jnp.float32),
            pltpu.VMEM((S,), jnp.float32),
            pltpu.VMEM((S,), jnp.int32),
            pltpu.VMEM((S,), jnp.int32),
            pltpu.VMEM((S,), jnp.float32),
            pltpu.SemaphoreType.DMA,
            pltpu.SemaphoreType.DMA,
        ],
    )(_sc_topk_body)
    actions, log_probs = sc_topk(q)
    ent = q[:, 3, :].reshape(B, S)

    topk_log_probs = jnp.zeros((B, S), jnp.float32)
    return (actions, topk_log_probs, log_probs, ent)


# speedup vs baseline: 1.0414x; 1.0022x over previous
import functools

import jax
import jax.numpy as jnp
from jax import lax
from jax.experimental import pallas as pl
from jax.experimental.pallas import tpu as pltpu
from jax.experimental.pallas import tpu_sc as plsc

B, S, D = 32, 4096, 768
K = S // 4
BS = 2048
NBLK = B * S // BS
RPB = S // BS

_I32_MIN = -2147483648
_I32_TOPMASK = 2147483647


def _stage1_body(x_ref, wt_ref, b_ref, q_ref):
    y = jax.lax.dot_general(wt_ref[...], x_ref[0],
                            (((1,), (1,)), ((), ())),
                            preferred_element_type=jnp.float32)
    l0 = y[0:1, :] + b_ref[0, 0]
    l1 = y[1:2, :] + b_ref[0, 1]
    score = l1 - l0
    m = jnp.maximum(l0, l1)
    lse = m + jnp.log(jnp.exp(l0 - m) + jnp.exp(l1 - m))
    lp0 = l0 - lse
    lp1 = l1 - lse
    ent = -(jnp.exp(lp0) * lp0 + jnp.exp(lp1) * lp1)
    q_ref[0] = jnp.concatenate(
        [score, lp0, lp1, ent, jnp.zeros((4, BS), jnp.float32)], axis=0)


_UNROLL = 16


def _sc_topk_body(q_hbm, act_hbm, lp_hbm, sc_v, p0_v, p1_v, key_v, act_v,
                  lpo_v, sem, sem_p):
    w = lax.axis_index("s") * 2 + lax.axis_index("c")

    copies = [
        pltpu.async_copy(q_hbm.at[2 * w, 0], sc_v.at[pl.ds(0, BS)], sem),
        pltpu.async_copy(q_hbm.at[2 * w + 1, 0], sc_v.at[pl.ds(BS, BS)], sem),
        pltpu.async_copy(q_hbm.at[2 * w, 1], p0_v.at[pl.ds(0, BS)], sem_p),
        pltpu.async_copy(q_hbm.at[2 * w + 1, 1], p0_v.at[pl.ds(BS, BS)],
                         sem_p),
        pltpu.async_copy(q_hbm.at[2 * w, 2], p1_v.at[pl.ds(0, BS)], sem_p),
        pltpu.async_copy(q_hbm.at[2 * w + 1, 2], p1_v.at[pl.ds(BS, BS)],
                         sem_p),
    ]
    copies[0].wait()
    copies[1].wait()

    def keys_body(j, carry):
        base = j * 16 * _UNROLL
        for u in range(_UNROLL):
            off = base + u * 16
            bits = lax.bitcast_convert_type(sc_v[pl.ds(off, 16)], jnp.int32)
            key_v[pl.ds(off, 16)] = jnp.where(
                bits < 0, bits ^ jnp.int32(_I32_TOPMASK), bits)
        return carry

    lax.fori_loop(0, S // (16 * _UNROLL), keys_body, jnp.int32(0))

    def count_matching(pred):
        def body(j, acc):
            base = j * 16 * _UNROLL
            for u in range(_UNROLL):
                k16 = key_v[pl.ds(base + u * 16, 16)]
                acc = acc + jnp.where(pred(k16, base + u * 16), 1, 0)
            return acc

        acc = lax.fori_loop(0, S // (16 * _UNROLL), body,
                            jnp.zeros((16,), jnp.int32))
        return jnp.sum(acc)

    def bit_body(i, carry):
        t_u, c_ge = carry
        cand = t_u | jnp.left_shift(jnp.int32(1), 31 - i)
        scand = cand ^ jnp.int32(_I32_MIN)
        cnt = count_matching(lambda k16, off: k16 >= scand)
        ok = cnt >= K
        return jnp.where(ok, cand, t_u), jnp.where(ok, cnt, c_ge)

    t_u, c_ge = lax.fori_loop(0, 32, bit_body,
                              (jnp.int32(0), jnp.int32(S)))
    thr = t_u ^ jnp.int32(_I32_MIN)

    c_gt = count_matching(lambda k16, off: k16 > thr)
    need = K - c_gt

    iota16 = lax.iota(jnp.int32, 16)

    def idx_descend():
        def idx_body(i, t_j):
            cand = t_j | jnp.left_shift(jnp.int32(1), 11 - i)
            cnt = count_matching(
                lambda k16, off: (k16 == thr) & (iota16 + off <= cand))
            return jnp.where(cnt <= need, cand, t_j)

        return lax.fori_loop(0, 12, idx_body, jnp.int32(0))

    t_j = lax.cond(c_ge - c_gt == need, lambda: jnp.int32(S), idx_descend)

    copies[2].wait()
    copies[3].wait()
    copies[4].wait()
    copies[5].wait()

    def fin_body(j, carry):
        base = j * 16 * _UNROLL
        for u in range(_UNROLL):
            off = base + u * 16
            k16 = key_v[pl.ds(off, 16)]
            m = (k16 > thr) | ((k16 == thr) & (iota16 + off <= t_j))
            act_v[pl.ds(off, 16)] = jnp.where(m, 1, 0)
            lpo_v[pl.ds(off, 16)] = jnp.where(
                m, p1_v[pl.ds(off, 16)], p0_v[pl.ds(off, 16)])
        return carry

    lax.fori_loop(0, S // (16 * _UNROLL), fin_body, jnp.int32(0))

    out_copies = [
        pltpu.async_copy(act_v, act_hbm.at[w], sem),
        pltpu.async_copy(lpo_v, lp_hbm.at[w], sem),
    ]
    out_copies[0].wait()
    out_copies[1].wait()


@jax.jit
def kernel(x, W, b):
    x3 = x.reshape(NBLK, BS, D)
    q = pl.pallas_call(
        _stage1_body,
        grid=(NBLK,),
        in_specs=[
            pl.BlockSpec((1, BS, D), lambda i: (i, 0, 0)),
            pl.BlockSpec((2, D), lambda i: (0, 0)),
            pl.BlockSpec((1, 2), lambda i: (0, 0)),
        ],
        out_specs=pl.BlockSpec((1, 8, BS), lambda i: (i, 0, 0)),
        out_shape=jax.ShapeDtypeStruct((NBLK, 8, BS), jnp.float32),
        compiler_params=pltpu.CompilerParams(
            dimension_semantics=("arbitrary",)),
    )(x3, W.T, b.reshape(1, 2))

    fdef = jax.ShapeDtypeStruct((B, S), jnp.float32)
    sc_topk = functools.partial(
        pl.kernel,
        mesh=plsc.VectorSubcoreMesh(core_axis_name="c", subcore_axis_name="s"),
        out_type=[jax.ShapeDtypeStruct((B, S), jnp.int32), fdef],
        compiler_params=pltpu.CompilerParams(needs_layout_passes=False),
        scratch_types=[
            pltpu.VMEM((S,), jnp.float32),
            pltpu.VMEM((S,), jnp.float32),
            pltpu.VMEM((S,), jnp.float32),
            pltpu.VMEM((S,), jnp.int32),
            pltpu.VMEM((S,), jnp.int32),
            pltpu.VMEM((S,), jnp.float32),
            pltpu.SemaphoreType.DMA,
            pltpu.SemaphoreType.DMA,
        ],
    )(_sc_topk_body)
    actions, log_probs = sc_topk(q)
    ent = q[:, 3, :].reshape(B, S)

    topk_log_probs = jnp.zeros((B, S), jnp.float32)
    return (actions, topk_log_probs, log_probs, ent)
